# baseline (speedup 1.0000x reference)
import functools

import jax
import jax.numpy as jnp
from jax import lax
from jax.experimental import pallas as pl
from jax.experimental.pallas import tpu as pltpu
from jax.experimental.pallas import tpu_sc as plsc

_NC = 2
_NS = 16
_NW = _NC * _NS
_CH = 128


def _sc_degree(dst2d, zeros16, n_acc):
    k = dst2d.shape[1]
    rows_sub = n_acc // _NS
    mesh = plsc.VectorSubcoreMesh(core_axis_name="c", subcore_axis_name="s")

    @functools.partial(
        pl.kernel,
        out_type=jax.ShapeDtypeStruct((_NC, n_acc, 16), jnp.float32),
        mesh=mesh,
        compiler_params=pltpu.CompilerParams(use_tc_tiling_on_sc=False),
        scratch_types=[
            pltpu.VMEM((k, _CH), jnp.int32),
            pltpu.VMEM((_CH, 16), jnp.float32),
            pltpu.VMEM_SHARED((n_acc, 16), jnp.float32),
        ],
    )
    def deg_kernel(dst_hbm, z_hbm, out_hbm, idx_v, ones_v, acc):
        cid = lax.axis_index("c")
        sid = lax.axis_index("s")
        wid = sid * _NC + cid

        @pl.loop(0, _CH)
        def _(i):
            ones_v[i, :] = jnp.full((16,), 1.0, jnp.float32)

        sub = pl.ds(sid * rows_sub, rows_sub)
        pltpu.sync_copy(z_hbm.at[sub], acc.at[sub])
        plsc.subcore_barrier()

        pltpu.sync_copy(dst_hbm.at[wid], idx_v)

        @pl.loop(0, k)
        def _(j):
            pltpu.sync_copy(ones_v, acc.at[idx_v.at[j]], add=True)

        plsc.subcore_barrier()
        pltpu.sync_copy(acc.at[sub], out_hbm.at[cid].at[sub])

    return deg_kernel(dst2d, zeros16)


_K0 = 56
_K1 = 104


def _sc_scatter(g, srcf, dstf, n_acc):
    rows_sub = n_acc // _NS
    nz_full = rows_sub // _CH
    nz_tail = rows_sub - nz_full * _CH
    kmax = max(_K0, _K1)
    mesh = plsc.VectorSubcoreMesh(core_axis_name="c", subcore_axis_name="s")

    @functools.partial(
        pl.kernel,
        out_type=jax.ShapeDtypeStruct((_NC, n_acc, 128), jnp.float32),
        mesh=mesh,
        scratch_types=[
            pltpu.VMEM((kmax, _CH), jnp.int32),
            pltpu.VMEM((kmax, _CH), jnp.int32),
            pltpu.VMEM((_CH, 128), jnp.float32),
            pltpu.VMEM_SHARED((n_acc, 128), jnp.float32),
            pltpu.SemaphoreType.DMA,
            pltpu.SemaphoreType.DMA,
        ],
    )
    def scat_kernel(g_hbm, src_hbm, dst_hbm, out_hbm,
                    isrc_v, idst_v, rows_v, acc, sem, sem_z):
        cid = lax.axis_index("c")
        sid = lax.axis_index("s")

        @pl.loop(0, _CH)
        def _(i):
            for q in range(8):
                rows_v[i, pl.ds(q * 16, 16)] = jnp.zeros((16,), jnp.float32)

        base = sid * rows_sub
        for d in range(nz_full):
            pltpu.async_copy(rows_v, acc.at[pl.ds(base + d * _CH, _CH)],
                             sem_z)
        if nz_tail:
            pltpu.async_copy(
                rows_v.at[pl.ds(0, nz_tail)],
                acc.at[pl.ds(base + nz_full * _CH, nz_tail)], sem_z)

        def edge_work(kc, estart):
            pltpu.sync_copy(src_hbm.at[pl.ds(estart, kc)],
                            isrc_v.at[pl.ds(0, kc)])
            pltpu.sync_copy(dst_hbm.at[pl.ds(estart, kc)],
                            idst_v.at[pl.ds(0, kc)])
            for d in range(nz_full):
                pltpu.make_async_copy(rows_v,
                                      acc.at[pl.ds(base + d * _CH, _CH)],
                                      sem_z).wait()
            if nz_tail:
                pltpu.make_async_copy(
                    rows_v.at[pl.ds(0, nz_tail)],
                    acc.at[pl.ds(base + nz_full * _CH, nz_tail)],
                    sem_z).wait()
            plsc.subcore_barrier()

            @pl.loop(0, kc)
            def _(j):
                pltpu.async_copy(g_hbm.at[isrc_v.at[j]], rows_v, sem).wait()
                pltpu.sync_copy(rows_v, acc.at[idst_v.at[j]], add=True)

        @pl.when(cid == 0)
        def _():
            edge_work(_K0, sid * (_K0 + _K1))

        @pl.when(cid == 1)
        def _():
            edge_work(_K1, sid * (_K0 + _K1) + _K0)

        plsc.subcore_barrier()
        sub = pl.ds(base, rows_sub)
        pltpu.sync_copy(acc.at[sub], out_hbm.at[cid].at[sub])

    return scat_kernel(g, srcf, dstf)


def _tc_matmul(x, w):
    n = x.shape[0]
    blk = 1000

    def body(x_ref, w_ref, o_ref):
        o_ref[...] = jnp.dot(x_ref[...], w_ref[...],
                             preferred_element_type=jnp.float32)

    return pl.pallas_call(
        body,
        grid=(n // blk,),
        in_specs=[
            pl.BlockSpec((blk, x.shape[1]), lambda i: (i, 0)),
            pl.BlockSpec(w.shape, lambda i: (0, 0)),
        ],
        out_specs=pl.BlockSpec((blk, w.shape[1]), lambda i: (i, 0)),
        out_shape=jax.ShapeDtypeStruct((n, w.shape[1]), jnp.float32),
    )(x, w)


def _tc_scale(h1, degp):
    n, d = h1.shape
    blk = 1000

    def body(h_ref, deg_ref, g_ref, dv_ref):
        deg = deg_ref[0][:, 0:1] + deg_ref[1][:, 0:1] + 1.0
        dv = jnp.broadcast_to(lax.rsqrt(deg), (blk, d))
        g_ref[...] = h_ref[...] * dv
        dv_ref[...] = dv

    return pl.pallas_call(
        body,
        grid=(n // blk,),
        in_specs=[
            pl.BlockSpec((blk, d), lambda i: (i, 0)),
            pl.BlockSpec((2, blk, 16), lambda i: (0, i, 0)),
        ],
        out_specs=[
            pl.BlockSpec((blk, d), lambda i: (i, 0)),
            pl.BlockSpec((blk, d), lambda i: (i, 0)),
        ],
        out_shape=[
            jax.ShapeDtypeStruct((n, d), jnp.float32),
            jax.ShapeDtypeStruct((n, d), jnp.float32),
        ],
    )(h1, degp)


def _tc_mid(s1, g1, dvb, b1, w2):
    n, d = g1.shape
    blk = 1000

    def body(s_ref, g_ref, dv_ref, b_ref, w_ref, o_ref):
        h = dv_ref[...] * (s_ref[0] + s_ref[1] + g_ref[...]) + b_ref[...]
        h = jnp.maximum(h, 0.0)
        h2 = jnp.dot(h, w_ref[...], preferred_element_type=jnp.float32)
        o_ref[...] = h2 * dv_ref[...]

    return pl.pallas_call(
        body,
        grid=(n // blk,),
        in_specs=[
            pl.BlockSpec((2, blk, d), lambda i: (0, i, 0)),
            pl.BlockSpec((blk, d), lambda i: (i, 0)),
            pl.BlockSpec((blk, d), lambda i: (i, 0)),
            pl.BlockSpec((1, d), lambda i: (0, 0)),
            pl.BlockSpec(w2.shape, lambda i: (0, 0)),
        ],
        out_specs=pl.BlockSpec((blk, d), lambda i: (i, 0)),
        out_shape=jax.ShapeDtypeStruct((n, d), jnp.float32),
    )(s1, g1, dvb, b1, w2)


def _tc_final(s2, g2, dvb, b2):
    n, d = g2.shape
    blk = 1000

    def body(s_ref, g_ref, dv_ref, b_ref, o_ref):
        o = dv_ref[...] * (s_ref[0] + s_ref[1] + g_ref[...]) + b_ref[...]
        m = jnp.max(o, axis=1, keepdims=True)
        z = o - m
        lse = jnp.log(jnp.sum(jnp.exp(z), axis=1, keepdims=True))
        o_ref[...] = z - lse

    return pl.pallas_call(
        body,
        grid=(n // blk,),
        in_specs=[
            pl.BlockSpec((2, blk, d), lambda i: (0, i, 0)),
            pl.BlockSpec((blk, d), lambda i: (i, 0)),
            pl.BlockSpec((blk, d), lambda i: (i, 0)),
            pl.BlockSpec((1, d), lambda i: (0, 0)),
        ],
        out_specs=pl.BlockSpec((blk, d), lambda i: (i, 0)),
        out_shape=jax.ShapeDtypeStruct((n, d), jnp.float32),
    )(s2, g2, dvb, b2)


@jax.jit
def kernel(x, edge_index, eigenvectors, W1, b1, W2, b2):
    n, d_in = x.shape
    e = edge_index.shape[1]

    rows_total = _NS * (_K0 + _K1)
    e_pad = rows_total * _CH
    assert e_pad >= e
    src_p = jnp.concatenate(
        [edge_index[0], jnp.zeros((e_pad - e,), jnp.int32)])
    dst_p = jnp.concatenate(
        [edge_index[1], jnp.full((e_pad - e,), n, jnp.int32)])
    srcf = src_p.reshape(rows_total, _CH)
    dstf = dst_p.reshape(rows_total, _CH)
    dst2d = dst_p.reshape(_NW, rows_total // _NW, _CH)

    n_acc = ((n + 1 + _NS * 8 - 1) // (_NS * 8)) * (_NS * 8)
    z16 = jnp.zeros((n_acc, 16), jnp.float32)

    degp = _sc_degree(dst2d, z16, n_acc)
    h1 = _tc_matmul(x, W1)
    g1, dvb = _tc_scale(h1, degp[:, :n, :])
    s1 = _sc_scatter(g1, srcf, dstf, n_acc)
    g2 = _tc_mid(s1[:, :n, :], g1, dvb, b1.reshape(1, -1), W2)
    s2 = _sc_scatter(g2, srcf, dstf, n_acc)
    return _tc_final(s2[:, :n, :], g2, dvb, b2.reshape(1, -1))


# speedup vs baseline: 1.9038x; 1.9038x over previous
import functools

import jax
import jax.numpy as jnp
from jax import lax
from jax.experimental import pallas as pl
from jax.experimental.pallas import tpu as pltpu
from jax.experimental.pallas import tpu_sc as plsc

_NC = 2
_NS = 16
_NW = _NC * _NS
_CH = 128


def _sc_degree(dst2d, zeros16, n_acc):
    k = dst2d.shape[1]
    rows_sub = n_acc // _NS
    mesh = plsc.VectorSubcoreMesh(core_axis_name="c", subcore_axis_name="s")

    @functools.partial(
        pl.kernel,
        out_type=jax.ShapeDtypeStruct((_NC, n_acc, 16), jnp.float32),
        mesh=mesh,
        compiler_params=pltpu.CompilerParams(use_tc_tiling_on_sc=False),
        scratch_types=[
            pltpu.VMEM((k, _CH), jnp.int32),
            pltpu.VMEM((_CH, 16), jnp.float32),
            pltpu.VMEM_SHARED((n_acc, 16), jnp.float32),
        ],
    )
    def deg_kernel(dst_hbm, z_hbm, out_hbm, idx_v, ones_v, acc):
        cid = lax.axis_index("c")
        sid = lax.axis_index("s")
        wid = sid * _NC + cid

        @pl.loop(0, _CH)
        def _(i):
            ones_v[i, :] = jnp.full((16,), 1.0, jnp.float32)

        sub = pl.ds(sid * rows_sub, rows_sub)
        pltpu.sync_copy(z_hbm.at[sub], acc.at[sub])
        plsc.subcore_barrier()

        pltpu.sync_copy(dst_hbm.at[wid], idx_v)

        @pl.loop(0, k)
        def _(j):
            pltpu.sync_copy(ones_v, acc.at[idx_v.at[j]], add=True)

        plsc.subcore_barrier()
        pltpu.sync_copy(acc.at[sub], out_hbm.at[cid].at[sub])

    return deg_kernel(dst2d, zeros16)


def _sc_scatter(g, src2d, dst2d, n_acc):
    k = dst2d.shape[1]
    rows_sub = n_acc // _NS
    nz_full = rows_sub // _CH
    nz_tail = rows_sub - nz_full * _CH
    mesh = plsc.VectorSubcoreMesh(core_axis_name="c", subcore_axis_name="s")

    @functools.partial(
        pl.kernel,
        out_type=jax.ShapeDtypeStruct((_NC, n_acc, 128), jnp.float32),
        mesh=mesh,
        scratch_types=[
            pltpu.VMEM((k, _CH), jnp.int32),
            pltpu.VMEM((k, _CH), jnp.int32),
            pltpu.VMEM((_CH, 128), jnp.float32),
            pltpu.VMEM_SHARED((n_acc, 128), jnp.float32),
            pltpu.SemaphoreType.DMA,
            pltpu.SemaphoreType.DMA,
        ],
    )
    def scat_kernel(g_hbm, src_hbm, dst_hbm, out_hbm,
                    isrc_v, idst_v, rows_v, acc, sem, sem_z):
        cid = lax.axis_index("c")
        sid = lax.axis_index("s")
        wid = sid * _NC + cid

        @pl.loop(0, _CH)
        def _(i):
            for q in range(8):
                rows_v[i, pl.ds(q * 16, 16)] = jnp.zeros((16,), jnp.float32)

        base = sid * rows_sub
        for d in range(nz_full):
            pltpu.async_copy(rows_v, acc.at[pl.ds(base + d * _CH, _CH)],
                             sem_z)
        if nz_tail:
            pltpu.async_copy(
                rows_v.at[pl.ds(0, nz_tail)],
                acc.at[pl.ds(base + nz_full * _CH, nz_tail)], sem_z)
        pltpu.sync_copy(src_hbm.at[wid], isrc_v)
        pltpu.sync_copy(dst_hbm.at[wid], idst_v)
        for d in range(nz_full):
            pltpu.make_async_copy(rows_v,
                                  acc.at[pl.ds(base + d * _CH, _CH)],
                                  sem_z).wait()
        if nz_tail:
            pltpu.make_async_copy(
                rows_v.at[pl.ds(0, nz_tail)],
                acc.at[pl.ds(base + nz_full * _CH, nz_tail)], sem_z).wait()
        plsc.subcore_barrier()

        @pl.loop(0, k)
        def _(j):
            pltpu.async_copy(g_hbm.at[isrc_v.at[j]], rows_v, sem).wait()
            pltpu.sync_copy(rows_v, acc.at[idst_v.at[j]], add=True)

        plsc.subcore_barrier()
        sub = pl.ds(base, rows_sub)
        pltpu.sync_copy(acc.at[sub], out_hbm.at[cid].at[sub])

    return scat_kernel(g, src2d, dst2d)


def _tc_matmul(x, w):
    n = x.shape[0]
    blk = 1000

    def body(x_ref, w_ref, o_ref):
        o_ref[...] = jnp.dot(x_ref[...], w_ref[...],
                             preferred_element_type=jnp.float32)

    return pl.pallas_call(
        body,
        grid=(n // blk,),
        in_specs=[
            pl.BlockSpec((blk, x.shape[1]), lambda i: (i, 0)),
            pl.BlockSpec(w.shape, lambda i: (0, 0)),
        ],
        out_specs=pl.BlockSpec((blk, w.shape[1]), lambda i: (i, 0)),
        out_shape=jax.ShapeDtypeStruct((n, w.shape[1]), jnp.float32),
    )(x, w)


def _tc_scale(h1, degp):
    n, d = h1.shape
    blk = 1000

    def body(h_ref, deg_ref, g_ref, dv_ref):
        deg = deg_ref[0][:, 0:1] + deg_ref[1][:, 0:1] + 1.0
        dv = jnp.broadcast_to(lax.rsqrt(deg), (blk, d))
        g_ref[...] = h_ref[...] * dv
        dv_ref[...] = dv

    return pl.pallas_call(
        body,
        grid=(n // blk,),
        in_specs=[
            pl.BlockSpec((blk, d), lambda i: (i, 0)),
            pl.BlockSpec((2, blk, 16), lambda i: (0, i, 0)),
        ],
        out_specs=[
            pl.BlockSpec((blk, d), lambda i: (i, 0)),
            pl.BlockSpec((blk, d), lambda i: (i, 0)),
        ],
        out_shape=[
            jax.ShapeDtypeStruct((n, d), jnp.float32),
            jax.ShapeDtypeStruct((n, d), jnp.float32),
        ],
    )(h1, degp)


def _tc_mid(s1, g1, dvb, b1, w2):
    n, d = g1.shape
    blk = 1000

    def body(s_ref, g_ref, dv_ref, b_ref, w_ref, o_ref):
        h = dv_ref[...] * (s_ref[0] + s_ref[1] + g_ref[...]) + b_ref[...]
        h = jnp.maximum(h, 0.0)
        h2 = jnp.dot(h, w_ref[...], preferred_element_type=jnp.float32)
        o_ref[...] = h2 * dv_ref[...]

    return pl.pallas_call(
        body,
        grid=(n // blk,),
        in_specs=[
            pl.BlockSpec((2, blk, d), lambda i: (0, i, 0)),
            pl.BlockSpec((blk, d), lambda i: (i, 0)),
            pl.BlockSpec((blk, d), lambda i: (i, 0)),
            pl.BlockSpec((1, d), lambda i: (0, 0)),
            pl.BlockSpec(w2.shape, lambda i: (0, 0)),
        ],
        out_specs=pl.BlockSpec((blk, d), lambda i: (i, 0)),
        out_shape=jax.ShapeDtypeStruct((n, d), jnp.float32),
    )(s1, g1, dvb, b1, w2)


def _tc_final(s2, g2, dvb, b2):
    n, d = g2.shape
    blk = 1000

    def body(s_ref, g_ref, dv_ref, b_ref, o_ref):
        o = dv_ref[...] * (s_ref[0] + s_ref[1] + g_ref[...]) + b_ref[...]
        m = jnp.max(o, axis=1, keepdims=True)
        z = o - m
        lse = jnp.log(jnp.sum(jnp.exp(z), axis=1, keepdims=True))
        o_ref[...] = z - lse

    return pl.pallas_call(
        body,
        grid=(n // blk,),
        in_specs=[
            pl.BlockSpec((2, blk, d), lambda i: (0, i, 0)),
            pl.BlockSpec((blk, d), lambda i: (i, 0)),
            pl.BlockSpec((blk, d), lambda i: (i, 0)),
            pl.BlockSpec((1, d), lambda i: (0, 0)),
        ],
        out_specs=pl.BlockSpec((blk, d), lambda i: (i, 0)),
        out_shape=jax.ShapeDtypeStruct((n, d), jnp.float32),
    )(s2, g2, dvb, b2)


@jax.jit
def kernel(x, edge_index, eigenvectors, W1, b1, W2, b2):
    n, d_in = x.shape
    e = edge_index.shape[1]

    k = -(-e // (_NW * _CH))
    e_pad = _NW * _CH * k
    src_p = jnp.concatenate(
        [edge_index[0], jnp.zeros((e_pad - e,), jnp.int32)])
    dst_p = jnp.concatenate(
        [edge_index[1], jnp.full((e_pad - e,), n, jnp.int32)])
    src2d = src_p.reshape(_NW, k, _CH)
    dst2d = dst_p.reshape(_NW, k, _CH)

    n_acc = ((n + 1 + _NS * 8 - 1) // (_NS * 8)) * (_NS * 8)
    z16 = jnp.zeros((n_acc, 16), jnp.float32)

    degp = _sc_degree(dst2d, z16, n_acc)
    h1 = _tc_matmul(x, W1)
    g1, dvb = _tc_scale(h1, degp[:, :n, :])
    s1 = _sc_scatter(g1, src2d, dst2d, n_acc)
    g2 = _tc_mid(s1[:, :n, :], g1, dvb, b1.reshape(1, -1), W2)
    s2 = _sc_scatter(g2, src2d, dst2d, n_acc)
    return _tc_final(s2[:, :n, :], g2, dvb, b2.reshape(1, -1))
